# baseline (speedup 1.0000x reference)
import functools

import jax
import jax.numpy as jnp
from jax.experimental import pallas as pl
from jax.experimental.pallas import tpu as pltpu
from jax.experimental.pallas import tpu_sc as plsc

H, W = 1024, 2048
RT = 128
NT = H // RT
TILE_N = RT * W
XSTEP = 2.0 / (W - 1)
YSTEP = 1.0 / (H - 1)
BIG = 2 ** 30
THRESH = 0.5
DIST_THRESH = 0.5
MIN_PIXEL = 160
MIN_INST_PIXEL = 160
MAX_INST = 200


def _chan_spec(c):
    return pl.BlockSpec((1, RT, W), lambda i, c=c: (c, i, 0))


def _img_spec():
    return pl.BlockSpec((RT, W), lambda i: (i, 0))


def _scalar_out_spec():
    return pl.BlockSpec((1, 1), lambda i: (0, 0), memory_space=pltpu.SMEM)


def _smem_spec():
    return pl.BlockSpec(memory_space=pltpu.SMEM)


def _scalar_sd(dtype):
    return jax.ShapeDtypeStruct((1, 1), dtype)


def _img_sd(dtype):
    return jax.ShapeDtypeStruct((H, W), dtype)


def _lidx():
    rows = jax.lax.broadcasted_iota(jnp.int32, (RT, W), 0)
    cols = jax.lax.broadcasted_iota(jnp.int32, (RT, W), 1)
    return rows * W + cols, rows, cols


def _emb(p0, p1, rows, cols, tile):
    x = cols.astype(jnp.float32) * XSTEP
    y = (tile * RT + rows).astype(jnp.float32) * YSTEP
    return jnp.tanh(p0) + x, jnp.tanh(p1) + y


def _pre_body(p5_ref, p6_ref, unc_ref, inst_ref, cnt_ref):
    @pl.when(pl.program_id(0) == 0)
    def _():
        cnt_ref[0, 0] = 0

    m = p6_ref[0] > p5_ref[0]
    mi = m.astype(jnp.int32)
    unc_ref[...] = mi
    inst_ref[...] = jnp.zeros((RT, W), jnp.int32)
    cnt_ref[0, 0] += jnp.sum(mi)


def _preamble(pred):
    return pl.pallas_call(
        _pre_body,
        grid=(NT,),
        in_specs=[_chan_spec(5), _chan_spec(6)],
        out_specs=[_img_spec(), _img_spec(), _scalar_out_spec()],
        out_shape=[_img_sd(jnp.int32), _img_sd(jnp.int32),
                   _scalar_sd(jnp.int32)],
    )(pred, pred)


def _b1_body(unc_ref, p0, p1, p2, p3, p5, p6,
             bv_ref, bi_ref, c0_ref, c1_ref, s0_ref, s1_ref):
    i = pl.program_id(0)

    @pl.when(i == 0)
    def _():
        bv_ref[0, 0] = -1.0
        bi_ref[0, 0] = 0
        c0_ref[0, 0] = 0.0
        c1_ref[0, 0] = 0.0
        s0_ref[0, 0] = 0.0
        s1_ref[0, 0] = 0.0

    seedm = jax.nn.sigmoid(p6[0] - p5[0])
    scores = seedm * unc_ref[...].astype(jnp.float32)
    m = jnp.max(scores)
    lidx, rows, cols = _lidx()
    loc = jnp.min(jnp.where(scores == m, lidx, BIG))
    sel = lidx == loc
    e0, e1 = _emb(p0[0], p1[0], rows, cols, i)
    c0 = jnp.sum(jnp.where(sel, e0, 0.0))
    c1 = jnp.sum(jnp.where(sel, e1, 0.0))
    s0 = jnp.sum(jnp.where(sel, jnp.exp(p2[0] * 10.0), 0.0))
    s1 = jnp.sum(jnp.where(sel, jnp.exp(p3[0] * 10.0), 0.0))

    @pl.when(m > bv_ref[0, 0])
    def _():
        bv_ref[0, 0] = m
        bi_ref[0, 0] = i * TILE_N + loc
        c0_ref[0, 0] = c0
        c1_ref[0, 0] = c1
        s0_ref[0, 0] = s0
        s1_ref[0, 0] = s1


def _b1(unc, pred):
    return pl.pallas_call(
        _b1_body,
        grid=(NT,),
        in_specs=[_img_spec()] + [_chan_spec(c) for c in (0, 1, 2, 3, 5, 6)],
        out_specs=[_scalar_out_spec()] * 6,
        out_shape=[_scalar_sd(jnp.float32), _scalar_sd(jnp.int32)]
        + [_scalar_sd(jnp.float32)] * 4,
    )(unc, *([pred] * 6))


def _b2_body(sv_ref, p0, p1, p2, p3, p4, p5, p6,
             prop_ref, n1_ref, bv_ref, bi_ref, c0_ref, c1_ref, s0_ref, s1_ref):
    i = pl.program_id(0)

    @pl.when(i == 0)
    def _():
        n1_ref[0, 0] = 0
        bv_ref[0, 0] = -1.0
        bi_ref[0, 0] = 0
        c0_ref[0, 0] = 0.0
        c1_ref[0, 0] = 0.0
        s0_ref[0, 0] = 0.0
        s1_ref[0, 0] = 0.0

    lidx, rows, cols = _lidx()
    e0, e1 = _emb(p0[0], p1[0], rows, cols, i)
    d0 = e0 - sv_ref[0]
    d1 = e1 - sv_ref[1]
    dist = jnp.exp(-(d0 * d0 * sv_ref[2] + d1 * d1 * sv_ref[3]))
    mask = p6[0] > p5[0]
    prop = (dist > DIST_THRESH) & mask
    prop_ref[...] = prop.astype(jnp.int32)
    n1_ref[0, 0] += jnp.sum(prop.astype(jnp.int32))

    sv = jnp.where(prop, jax.nn.sigmoid(p4[0]), 0.0)
    m = jnp.max(sv)
    loc = jnp.min(jnp.where(sv == m, lidx, BIG))
    sel = lidx == loc
    c0 = jnp.sum(jnp.where(sel, e0, 0.0))
    c1 = jnp.sum(jnp.where(sel, e1, 0.0))
    s0 = jnp.sum(jnp.where(sel, jnp.exp(p2[0] * 10.0), 0.0))
    s1 = jnp.sum(jnp.where(sel, jnp.exp(p3[0] * 10.0), 0.0))

    @pl.when(m > bv_ref[0, 0])
    def _():
        bv_ref[0, 0] = m
        bi_ref[0, 0] = i * TILE_N + loc
        c0_ref[0, 0] = c0
        c1_ref[0, 0] = c1
        s0_ref[0, 0] = s0
        s1_ref[0, 0] = s1


def _b2(center_sig, pred):
    return pl.pallas_call(
        _b2_body,
        grid=(NT,),
        in_specs=[_smem_spec()] + [_chan_spec(c) for c in range(7)],
        out_specs=[_img_spec()] + [_scalar_out_spec()] * 7,
        out_shape=[_img_sd(jnp.int32), _scalar_sd(jnp.int32),
                   _scalar_sd(jnp.float32), _scalar_sd(jnp.int32)]
        + [_scalar_sd(jnp.float32)] * 4,
    )(center_sig, *([pred] * 7))


def _b3_body(sv_ref, iv_ref, unc_ref, p0, p1, p5, p6,
             prop_ref, n2_ref, r_ref, cs1_ref, cs2_ref):
    i = pl.program_id(0)

    @pl.when(i == 0)
    def _():
        n2_ref[0, 0] = 0
        r_ref[0, 0] = 0
        cs1_ref[0, 0] = 0
        cs2_ref[0, 0] = 0

    lidx, rows, cols = _lidx()
    e0, e1 = _emb(p0[0], p1[0], rows, cols, i)
    d0 = e0 - sv_ref[0]
    d1 = e1 - sv_ref[1]
    dist = jnp.exp(-(d0 * d0 * sv_ref[2] + d1 * d1 * sv_ref[3]))
    mask = p6[0] > p5[0]
    prop = ((dist > DIST_THRESH) & mask).astype(jnp.int32)
    prop_ref[...] = prop
    u = unc_ref[...]
    pu = prop * u
    n2_ref[0, 0] += jnp.sum(prop)
    r_ref[0, 0] += jnp.sum(pu)
    gidx = lidx + i * TILE_N
    cs1_ref[0, 0] += jnp.sum(jnp.where(gidx == iv_ref[0], pu, 0))
    cs2_ref[0, 0] += jnp.sum(jnp.where(gidx == iv_ref[1], pu, 0))


def _b3(center_sig2, seeds, unc, pred):
    return pl.pallas_call(
        _b3_body,
        grid=(NT,),
        in_specs=[_smem_spec(), _smem_spec(), _img_spec()]
        + [_chan_spec(c) for c in (0, 1, 5, 6)],
        out_specs=[_img_spec()] + [_scalar_out_spec()] * 4,
        out_shape=[_img_sd(jnp.int32)] + [_scalar_sd(jnp.int32)] * 4,
    )(center_sig2, seeds, unc, *([pred] * 4))


def _b4_body(iv_ref, unc_ref, inst_ref, p1_ref, p2_ref,
             unc_out, inst_out, sum_ref):
    i = pl.program_id(0)

    @pl.when(i == 0)
    def _():
        sum_ref[0, 0] = 0

    seed = iv_ref[0]
    seed2 = iv_ref[1]
    count = iv_ref[2]
    broke = iv_ref[3] != 0
    big1 = iv_ref[4] != 0
    assign = iv_ref[5] != 0

    lidx, _, _ = _lidx()
    gidx = lidx + i * TILE_N
    u = unc_ref[...]
    u1 = jnp.where(gidx == seed, 0, u)
    u2 = jnp.where(gidx == seed2, 0, u1)
    prop1 = p1_ref[...]
    prop2 = p2_ref[...]
    fp = jnp.where(big1, prop2, prop1)
    umid = jnp.where(big1, u2, u1)
    unew = jnp.where(fp != 0, 0, umid)
    unew = jnp.where(broke, u, unew)
    inst = inst_ref[...]
    inew = jnp.where(assign & (prop2 != 0), count, inst)
    inew = jnp.where(broke, inst, inew)
    unc_out[...] = unew
    inst_out[...] = inew
    sum_ref[0, 0] += jnp.sum(unew)


def _b4(ivec, unc, inst, prop1, prop2):
    return pl.pallas_call(
        _b4_body,
        grid=(NT,),
        in_specs=[_smem_spec()] + [_img_spec()] * 4,
        out_specs=[_img_spec(), _img_spec(), _scalar_out_spec()],
        out_shape=[_img_sd(jnp.int32), _img_sd(jnp.int32),
                   _scalar_sd(jnp.int32)],
    )(ivec, unc, inst, prop1, prop2)


NW = 32
PER_W = (H * W) // NW
NB = 256


def _sc_hist_body(inst_hbm, out_hbm, buf_v, tab_v, res_v):
    c = jax.lax.axis_index("c")
    s = jax.lax.axis_index("s")
    wid = s * 2 + c
    z16 = jnp.zeros((16,), jnp.int32)
    for r in range(16):
        for g in range(NB // 16):
            tab_v[r, pl.ds(g * 16, 16)] = z16

    pltpu.sync_copy(inst_hbm.at[pl.ds(wid * PER_W, PER_W)], buf_v)
    lane = jax.lax.iota(jnp.int32, 16)
    ones = jnp.ones((16,), jnp.int32)

    def body(j, carry):
        v = plsc.load_gather(buf_v, [lane + j * 16])
        plsc.addupdate_scatter(tab_v, [lane, v], ones)
        return carry

    jax.lax.fori_loop(0, PER_W // 16, body, 0)

    for g in range(NB // 16):
        acc = z16
        for r in range(16):
            acc = acc + tab_v[r, pl.ds(g * 16, 16)]
        res_v[pl.ds(g * 16, 16)] = acc
    pltpu.sync_copy(res_v, out_hbm.at[wid])


def _sc_hist(inst):
    mesh = plsc.VectorSubcoreMesh(core_axis_name="c", subcore_axis_name="s",
                                  num_cores=2, num_subcores=16)
    f = pl.kernel(
        _sc_hist_body,
        out_type=jax.ShapeDtypeStruct((NW, NB), jnp.int32),
        mesh=mesh,
        compiler_params=pltpu.CompilerParams(needs_layout_passes=False),
        scratch_types=[pltpu.VMEM((PER_W,), jnp.int32),
                       pltpu.VMEM((16, NB), jnp.int32),
                       pltpu.VMEM((NB,), jnp.int32)],
    )
    return f(inst.reshape(-1))


def _relabel_body(rm_ref, nrm_ref, inst_ref, out_ref):
    t = inst_ref[...]

    def body(j, acc):
        return jnp.where(t == rm_ref[j], 0, acc)

    res = jax.lax.fori_loop(0, nrm_ref[0], body, t)
    out_ref[...] = res.astype(jnp.uint8)


def _relabel(rm, nrm, inst):
    return pl.pallas_call(
        _relabel_body,
        grid=(NT,),
        in_specs=[_smem_spec(), _smem_spec(), _img_spec()],
        out_specs=_img_spec(),
        out_shape=_img_sd(jnp.uint8),
    )(rm, nrm, inst)


@functools.partial(jax.jit, static_argnames=())
def kernel(prediction):
    pred = prediction[0]

    unc0, inst0, cnt = _preamble(pred)
    unc_sum0 = jnp.sum(cnt)

    def cond_fn(state):
        unc, inst, sizes, count, done, unc_sum = state
        return (~done) & (unc_sum > MIN_PIXEL) & (count < MAX_INST)

    def body_fn(state):
        unc, inst, sizes, count, done, unc_sum = state
        bv, bi, c0, c1, s0, s1 = _b1(unc, pred)
        seed_score = bv[0, 0]
        seed = bi[0, 0]
        broke = seed_score < THRESH
        csig = jnp.stack([c0[0, 0], c1[0, 0], s0[0, 0], s1[0, 0]])
        prop1, n1r, bv2, bi2, c20, c21, s20, s21 = _b2(csig, pred)
        n1 = n1r[0, 0]
        big1 = n1 > MIN_INST_PIXEL
        seed2 = bi2[0, 0]
        csig2 = jnp.stack([c20[0, 0], c21[0, 0], s20[0, 0], s21[0, 0]])
        seeds = jnp.stack([seed, seed2])
        prop2, n2r, rr, cs1, cs2 = _b3(csig2, seeds, unc, pred)
        n2 = n2r[0, 0]
        big2 = n2 > MIN_INST_PIXEL
        inner = rr[0, 0] - cs1[0, 0] - jnp.where(seed2 != seed, cs2[0, 0], 0)
        ratio_ok = 2 * inner > n2
        assign = big1 & big2 & ratio_ok
        ivec = jnp.stack([seed, seed2, count,
                          broke.astype(jnp.int32), big1.astype(jnp.int32),
                          assign.astype(jnp.int32)])
        unc_new, inst_new, sum_new = _b4(ivec, unc, inst, prop1, prop2)
        keep = jnp.logical_and(assign, ~broke)
        sizes_new = jnp.where(keep, sizes.at[count].set(n2), sizes)
        count_new = count + jnp.where(keep, 1, 0)
        return (unc_new, inst_new, sizes_new, count_new, broke,
                sum_new[0, 0])

    state0 = (unc0, inst0, jnp.zeros((MAX_INST,), jnp.int32),
              jnp.int32(1), jnp.asarray(False), unc_sum0)
    unc, inst, sizes, count, done, unc_sum = jax.lax.while_loop(
        cond_fn, body_fn, state0)

    nowp = jax.lax.cond(count > 1, lambda: _sc_hist(inst),
                        lambda: jnp.zeros((NW, NB), jnp.int32))
    now = jnp.sum(nowp, axis=0)[:MAX_INST]
    prev = sizes
    remove = (now > 0) & (prev != now) & (
        (now < MIN_INST_PIXEL * 3) | (2 * now < prev))
    remove = remove.at[0].set(False)
    rm = jnp.sort(jnp.where(remove, jnp.arange(MAX_INST, dtype=jnp.int32),
                            MAX_INST))
    nrm = jnp.sum(remove.astype(jnp.int32)).reshape(1)
    out = _relabel(rm, nrm, inst)
    return out.reshape(1, H, W)


# speedup vs baseline: 1.5081x; 1.5081x over previous
import functools

import jax
import jax.numpy as jnp
from jax.experimental import pallas as pl
from jax.experimental.pallas import tpu as pltpu
from jax.experimental.pallas import tpu_sc as plsc

H, W = 1024, 2048
RT = 128
NT = H // RT
TILE_N = RT * W
XSTEP = 2.0 / (W - 1)
YSTEP = 1.0 / (H - 1)
BIG = 2 ** 30
THRESH = 0.5
DIST_THRESH = 0.5
MIN_PIXEL = 160
MIN_INST_PIXEL = 160
MAX_INST = 200


def _chan_spec(c):
    return pl.BlockSpec((1, RT, W), lambda i, c=c: (c, i, 0))


def _img_spec():
    return pl.BlockSpec((RT, W), lambda i: (i, 0))


def _scalar_out_spec():
    return pl.BlockSpec((1, 1), lambda i: (0, 0), memory_space=pltpu.SMEM)


def _smem_spec():
    return pl.BlockSpec(memory_space=pltpu.SMEM)


def _scalar_sd(dtype):
    return jax.ShapeDtypeStruct((1, 1), dtype)


def _img_sd(dtype):
    return jax.ShapeDtypeStruct((H, W), dtype)


def _lidx():
    rows = jax.lax.broadcasted_iota(jnp.int32, (RT, W), 0)
    cols = jax.lax.broadcasted_iota(jnp.int32, (RT, W), 1)
    return rows * W + cols, rows, cols


def _emb(p0, p1, rows, cols, tile):
    x = cols.astype(jnp.float32) * XSTEP
    y = (tile * RT + rows).astype(jnp.float32) * YSTEP
    return jnp.tanh(p0) + x, jnp.tanh(p1) + y


def _pre_body(p5_ref, p6_ref, unc_ref, inst_ref, cnt_ref):
    @pl.when(pl.program_id(0) == 0)
    def _():
        cnt_ref[0, 0] = 0

    m = p6_ref[0] > p5_ref[0]
    mi = m.astype(jnp.int32)
    unc_ref[...] = mi
    inst_ref[...] = jnp.zeros((RT, W), jnp.int32)
    cnt_ref[0, 0] += jnp.sum(mi)


def _preamble(pred):
    return pl.pallas_call(
        _pre_body,
        grid=(NT,),
        in_specs=[_chan_spec(5), _chan_spec(6)],
        out_specs=[_img_spec(), _img_spec(), _scalar_out_spec()],
        out_shape=[_img_sd(jnp.int32), _img_sd(jnp.int32),
                   _scalar_sd(jnp.int32)],
    )(pred, pred)


def _b1_body(unc_ref, p0, p1, p2, p3, p5, p6,
             bv_ref, bi_ref, c0_ref, c1_ref, s0_ref, s1_ref):
    i = pl.program_id(0)

    @pl.when(i == 0)
    def _():
        bv_ref[0, 0] = -1.0
        bi_ref[0, 0] = 0
        c0_ref[0, 0] = 0.0
        c1_ref[0, 0] = 0.0
        s0_ref[0, 0] = 0.0
        s1_ref[0, 0] = 0.0

    seedm = jax.nn.sigmoid(p6[0] - p5[0])
    scores = seedm * unc_ref[...].astype(jnp.float32)
    m = jnp.max(scores)
    lidx, rows, cols = _lidx()
    loc = jnp.min(jnp.where(scores == m, lidx, BIG))
    sel = lidx == loc
    e0, e1 = _emb(p0[0], p1[0], rows, cols, i)
    c0 = jnp.sum(jnp.where(sel, e0, 0.0))
    c1 = jnp.sum(jnp.where(sel, e1, 0.0))
    s0 = jnp.sum(jnp.where(sel, jnp.exp(p2[0] * 10.0), 0.0))
    s1 = jnp.sum(jnp.where(sel, jnp.exp(p3[0] * 10.0), 0.0))

    @pl.when(m > bv_ref[0, 0])
    def _():
        bv_ref[0, 0] = m
        bi_ref[0, 0] = i * TILE_N + loc
        c0_ref[0, 0] = c0
        c1_ref[0, 0] = c1
        s0_ref[0, 0] = s0
        s1_ref[0, 0] = s1


def _b1(unc, pred):
    return pl.pallas_call(
        _b1_body,
        grid=(NT,),
        in_specs=[_img_spec()] + [_chan_spec(c) for c in (0, 1, 2, 3, 5, 6)],
        out_specs=[_scalar_out_spec()] * 6,
        out_shape=[_scalar_sd(jnp.float32), _scalar_sd(jnp.int32)]
        + [_scalar_sd(jnp.float32)] * 4,
    )(unc, *([pred] * 6))


def _b2_body(sv_ref, p0, p1, p2, p3, p4, p5, p6,
             prop_ref, n1_ref, bv_ref, bi_ref, c0_ref, c1_ref, s0_ref, s1_ref):
    i = pl.program_id(0)

    @pl.when(i == 0)
    def _():
        n1_ref[0, 0] = 0
        bv_ref[0, 0] = -1.0
        bi_ref[0, 0] = 0
        c0_ref[0, 0] = 0.0
        c1_ref[0, 0] = 0.0
        s0_ref[0, 0] = 0.0
        s1_ref[0, 0] = 0.0

    lidx, rows, cols = _lidx()
    e0, e1 = _emb(p0[0], p1[0], rows, cols, i)
    d0 = e0 - sv_ref[0]
    d1 = e1 - sv_ref[1]
    dist = jnp.exp(-(d0 * d0 * sv_ref[2] + d1 * d1 * sv_ref[3]))
    mask = p6[0] > p5[0]
    prop = (dist > DIST_THRESH) & mask
    prop_ref[...] = prop.astype(jnp.int32)
    n1_ref[0, 0] += jnp.sum(prop.astype(jnp.int32))

    sv = jnp.where(prop, jax.nn.sigmoid(p4[0]), 0.0)
    m = jnp.max(sv)
    loc = jnp.min(jnp.where(sv == m, lidx, BIG))
    sel = lidx == loc
    c0 = jnp.sum(jnp.where(sel, e0, 0.0))
    c1 = jnp.sum(jnp.where(sel, e1, 0.0))
    s0 = jnp.sum(jnp.where(sel, jnp.exp(p2[0] * 10.0), 0.0))
    s1 = jnp.sum(jnp.where(sel, jnp.exp(p3[0] * 10.0), 0.0))

    @pl.when(m > bv_ref[0, 0])
    def _():
        bv_ref[0, 0] = m
        bi_ref[0, 0] = i * TILE_N + loc
        c0_ref[0, 0] = c0
        c1_ref[0, 0] = c1
        s0_ref[0, 0] = s0
        s1_ref[0, 0] = s1


def _b2(center_sig, pred):
    return pl.pallas_call(
        _b2_body,
        grid=(NT,),
        in_specs=[_smem_spec()] + [_chan_spec(c) for c in range(7)],
        out_specs=[_img_spec()] + [_scalar_out_spec()] * 7,
        out_shape=[_img_sd(jnp.int32), _scalar_sd(jnp.int32),
                   _scalar_sd(jnp.float32), _scalar_sd(jnp.int32)]
        + [_scalar_sd(jnp.float32)] * 4,
    )(center_sig, *([pred] * 7))


def _b3_body(sv_ref, iv_ref, unc_ref, p0, p1, p5, p6,
             prop_ref, n2_ref, r_ref, cs1_ref, cs2_ref):
    i = pl.program_id(0)

    @pl.when(i == 0)
    def _():
        n2_ref[0, 0] = 0
        r_ref[0, 0] = 0
        cs1_ref[0, 0] = 0
        cs2_ref[0, 0] = 0

    lidx, rows, cols = _lidx()
    e0, e1 = _emb(p0[0], p1[0], rows, cols, i)
    d0 = e0 - sv_ref[0]
    d1 = e1 - sv_ref[1]
    dist = jnp.exp(-(d0 * d0 * sv_ref[2] + d1 * d1 * sv_ref[3]))
    mask = p6[0] > p5[0]
    prop = ((dist > DIST_THRESH) & mask).astype(jnp.int32)
    prop_ref[...] = prop
    u = unc_ref[...]
    pu = prop * u
    n2_ref[0, 0] += jnp.sum(prop)
    r_ref[0, 0] += jnp.sum(pu)
    gidx = lidx + i * TILE_N
    cs1_ref[0, 0] += jnp.sum(jnp.where(gidx == iv_ref[0], pu, 0))
    cs2_ref[0, 0] += jnp.sum(jnp.where(gidx == iv_ref[1], pu, 0))


def _b3(center_sig2, seeds, unc, pred):
    return pl.pallas_call(
        _b3_body,
        grid=(NT,),
        in_specs=[_smem_spec(), _smem_spec(), _img_spec()]
        + [_chan_spec(c) for c in (0, 1, 5, 6)],
        out_specs=[_img_spec()] + [_scalar_out_spec()] * 4,
        out_shape=[_img_sd(jnp.int32)] + [_scalar_sd(jnp.int32)] * 4,
    )(center_sig2, seeds, unc, *([pred] * 4))


def _b4_body(iv_ref, unc_ref, inst_ref, p1_ref, p2_ref,
             unc_out, inst_out, sum_ref):
    i = pl.program_id(0)

    @pl.when(i == 0)
    def _():
        sum_ref[0, 0] = 0

    seed = iv_ref[0]
    seed2 = iv_ref[1]
    count = iv_ref[2]
    broke = iv_ref[3] != 0
    big1 = iv_ref[4] != 0
    assign = iv_ref[5] != 0

    lidx, _, _ = _lidx()
    gidx = lidx + i * TILE_N
    u = unc_ref[...]
    u1 = jnp.where(gidx == seed, 0, u)
    u2 = jnp.where(gidx == seed2, 0, u1)
    prop1 = p1_ref[...]
    prop2 = p2_ref[...]
    fp = jnp.where(big1, prop2, prop1)
    umid = jnp.where(big1, u2, u1)
    unew = jnp.where(fp != 0, 0, umid)
    unew = jnp.where(broke, u, unew)
    inst = inst_ref[...]
    inew = jnp.where(assign & (prop2 != 0), count, inst)
    inew = jnp.where(broke, inst, inew)
    unc_out[...] = unew
    inst_out[...] = inew
    sum_ref[0, 0] += jnp.sum(unew)


def _b4(ivec, unc, inst, prop1, prop2):
    return pl.pallas_call(
        _b4_body,
        grid=(NT,),
        in_specs=[_smem_spec()] + [_img_spec()] * 4,
        out_specs=[_img_spec(), _img_spec(), _scalar_out_spec()],
        out_shape=[_img_sd(jnp.int32), _img_sd(jnp.int32),
                   _scalar_sd(jnp.int32)],
    )(ivec, unc, inst, prop1, prop2)


NW = 32
PER_W = (H * W) // NW
NB = 256


def _sc_hist_body(inst_hbm, out_hbm, buf_v, tab_v, res_v):
    c = jax.lax.axis_index("c")
    s = jax.lax.axis_index("s")
    wid = s * 2 + c
    z16 = jnp.zeros((16,), jnp.int32)
    for r in range(16):
        for g in range(NB // 16):
            tab_v[r, pl.ds(g * 16, 16)] = z16

    pltpu.sync_copy(inst_hbm.at[pl.ds(wid * PER_W, PER_W)], buf_v)
    lane = jax.lax.iota(jnp.int32, 16)
    ones = jnp.ones((16,), jnp.int32)

    def body(j, carry):
        v = plsc.load_gather(buf_v, [lane + j * 16])
        plsc.addupdate_scatter(tab_v, [lane, v], ones)
        return carry

    jax.lax.fori_loop(0, PER_W // 16, body, 0)

    for g in range(NB // 16):
        acc = z16
        for r in range(16):
            acc = acc + tab_v[r, pl.ds(g * 16, 16)]
        res_v[pl.ds(g * 16, 16)] = acc
    pltpu.sync_copy(res_v, out_hbm.at[wid])


def _tc_hist_body(inst_ref, now_ref):
    i = pl.program_id(0)

    @pl.when(i == 0)
    def _():
        now_ref[...] = jnp.zeros((1, NB), jnp.int32)

    t = inst_ref[...]
    lane = jax.lax.broadcasted_iota(jnp.int32, (1, NB), 1)

    def body(b, _):
        c = jnp.sum((t == b).astype(jnp.int32))
        now_ref[...] += jnp.where(lane == b, c, 0)
        return 0

    jax.lax.fori_loop(1, MAX_INST, body, 0)


def _tc_hist(inst):
    return pl.pallas_call(
        _tc_hist_body,
        grid=(NT,),
        in_specs=[_img_spec()],
        out_specs=pl.BlockSpec((1, NB), lambda i: (0, 0)),
        out_shape=jax.ShapeDtypeStruct((1, NB), jnp.int32),
    )(inst)


def _sc_hist(inst):
    mesh = plsc.VectorSubcoreMesh(core_axis_name="c", subcore_axis_name="s",
                                  num_cores=2, num_subcores=16)
    f = pl.kernel(
        _sc_hist_body,
        out_type=jax.ShapeDtypeStruct((NW, NB), jnp.int32),
        mesh=mesh,
        compiler_params=pltpu.CompilerParams(needs_layout_passes=False),
        scratch_types=[pltpu.VMEM((PER_W,), jnp.int32),
                       pltpu.VMEM((16, NB), jnp.int32),
                       pltpu.VMEM((NB,), jnp.int32)],
    )
    return f(inst.reshape(-1))


def _relabel_body(rm_ref, nrm_ref, inst_ref, out_ref):
    t = inst_ref[...]

    def body(j, acc):
        return jnp.where(t == rm_ref[j], 0, acc)

    res = jax.lax.fori_loop(0, nrm_ref[0], body, t)
    out_ref[...] = res.astype(jnp.uint8)


def _relabel(rm, nrm, inst):
    return pl.pallas_call(
        _relabel_body,
        grid=(NT,),
        in_specs=[_smem_spec(), _smem_spec(), _img_spec()],
        out_specs=_img_spec(),
        out_shape=_img_sd(jnp.uint8),
    )(rm, nrm, inst)


@functools.partial(jax.jit, static_argnames=())
def kernel(prediction):
    pred = prediction[0]

    unc0, inst0, cnt = _preamble(pred)
    unc_sum0 = jnp.sum(cnt)

    def cond_fn(state):
        unc, inst, sizes, count, done, unc_sum = state
        return (~done) & (unc_sum > MIN_PIXEL) & (count < MAX_INST)

    def body_fn(state):
        unc, inst, sizes, count, done, unc_sum = state
        bv, bi, c0, c1, s0, s1 = _b1(unc, pred)
        seed_score = bv[0, 0]
        seed = bi[0, 0]
        broke = seed_score < THRESH
        csig = jnp.stack([c0[0, 0], c1[0, 0], s0[0, 0], s1[0, 0]])
        prop1, n1r, bv2, bi2, c20, c21, s20, s21 = _b2(csig, pred)
        n1 = n1r[0, 0]
        big1 = n1 > MIN_INST_PIXEL
        seed2 = bi2[0, 0]
        csig2 = jnp.stack([c20[0, 0], c21[0, 0], s20[0, 0], s21[0, 0]])
        seeds = jnp.stack([seed, seed2])
        prop2, n2r, rr, cs1, cs2 = _b3(csig2, seeds, unc, pred)
        n2 = n2r[0, 0]
        big2 = n2 > MIN_INST_PIXEL
        inner = rr[0, 0] - cs1[0, 0] - jnp.where(seed2 != seed, cs2[0, 0], 0)
        ratio_ok = 2 * inner > n2
        assign = big1 & big2 & ratio_ok
        ivec = jnp.stack([seed, seed2, count,
                          broke.astype(jnp.int32), big1.astype(jnp.int32),
                          assign.astype(jnp.int32)])
        unc_new, inst_new, sum_new = _b4(ivec, unc, inst, prop1, prop2)
        keep = jnp.logical_and(assign, ~broke)
        sizes_new = jnp.where(keep, sizes.at[count].set(n2), sizes)
        count_new = count + jnp.where(keep, 1, 0)
        return (unc_new, inst_new, sizes_new, count_new, broke,
                sum_new[0, 0])

    state0 = (unc0, inst0, jnp.zeros((MAX_INST,), jnp.int32),
              jnp.int32(1), jnp.asarray(False), unc_sum0)
    unc, inst, sizes, count, done, unc_sum = jax.lax.while_loop(
        cond_fn, body_fn, state0)

    nowp = jax.lax.cond(count > 1, lambda: _tc_hist(inst),
                        lambda: jnp.zeros((1, NB), jnp.int32))
    now = nowp[0, :MAX_INST]
    prev = sizes
    remove = (now > 0) & (prev != now) & (
        (now < MIN_INST_PIXEL * 3) | (2 * now < prev))
    remove = remove.at[0].set(False)
    rm = jnp.sort(jnp.where(remove, jnp.arange(MAX_INST, dtype=jnp.int32),
                            MAX_INST))
    nrm = jnp.sum(remove.astype(jnp.int32)).reshape(1)
    out = _relabel(rm, nrm, inst)
    return out.reshape(1, H, W)


# speedup vs baseline: 1.8784x; 1.2456x over previous
import functools

import jax
import jax.numpy as jnp
from jax.experimental import pallas as pl
from jax.experimental.pallas import tpu as pltpu
from jax.experimental.pallas import tpu_sc as plsc

H, W = 1024, 2048
RT = 128
NT = H // RT
TILE_N = RT * W
XSTEP = 2.0 / (W - 1)
YSTEP = 1.0 / (H - 1)
BIG = 2 ** 30
THRESH = 0.5
DIST_THRESH = 0.5
MIN_PIXEL = 160
MIN_INST_PIXEL = 160
MAX_INST = 200


def _chan_spec(c):
    return pl.BlockSpec((1, RT, W), lambda i, c=c: (c, i, 0))


def _img_spec():
    return pl.BlockSpec((RT, W), lambda i: (i, 0))


def _scalar_out_spec():
    return pl.BlockSpec((1, 1), lambda i: (0, 0), memory_space=pltpu.SMEM)


def _smem_spec():
    return pl.BlockSpec(memory_space=pltpu.SMEM)


def _scalar_sd(dtype):
    return jax.ShapeDtypeStruct((1, 1), dtype)


def _img_sd(dtype):
    return jax.ShapeDtypeStruct((H, W), dtype)


def _lidx():
    rows = jax.lax.broadcasted_iota(jnp.int32, (RT, W), 0)
    cols = jax.lax.broadcasted_iota(jnp.int32, (RT, W), 1)
    return rows * W + cols, rows, cols


def _emb(p0, p1, rows, cols, tile):
    x = cols.astype(jnp.float32) * XSTEP
    y = (tile * RT + rows).astype(jnp.float32) * YSTEP
    return jnp.tanh(p0) + x, jnp.tanh(p1) + y


def _pre_body(p5_ref, p6_ref, unc_ref, inst_ref, cnt_ref):
    @pl.when(pl.program_id(0) == 0)
    def _():
        cnt_ref[0, 0] = 0

    m = p6_ref[0] > p5_ref[0]
    unc_ref[...] = m.astype(jnp.int8)
    inst_ref[...] = jnp.zeros((RT, W), jnp.uint8)
    cnt_ref[0, 0] += jnp.sum(m.astype(jnp.int32))


def _preamble(pred):
    return pl.pallas_call(
        _pre_body,
        grid=(NT,),
        in_specs=[_chan_spec(5), _chan_spec(6)],
        out_specs=[_img_spec(), _img_spec(), _scalar_out_spec()],
        out_shape=[_img_sd(jnp.int8), _img_sd(jnp.uint8),
                   _scalar_sd(jnp.int32)],
    )(pred, pred)


def _b1_body(unc_ref, p0, p1, p2, p3, p5, p6,
             bv_ref, bi_ref, c0_ref, c1_ref, s0_ref, s1_ref):
    i = pl.program_id(0)

    @pl.when(i == 0)
    def _():
        bv_ref[0, 0] = -1.0
        bi_ref[0, 0] = 0
        c0_ref[0, 0] = 0.0
        c1_ref[0, 0] = 0.0
        s0_ref[0, 0] = 0.0
        s1_ref[0, 0] = 0.0

    seedm = jax.nn.sigmoid(p6[0] - p5[0])
    scores = seedm * unc_ref[...].astype(jnp.float32)
    m = jnp.max(scores)
    lidx, rows, cols = _lidx()
    loc = jnp.min(jnp.where(scores == m, lidx, BIG))
    sel = lidx == loc
    e0, e1 = _emb(p0[0], p1[0], rows, cols, i)
    c0 = jnp.sum(jnp.where(sel, e0, 0.0))
    c1 = jnp.sum(jnp.where(sel, e1, 0.0))
    s0 = jnp.sum(jnp.where(sel, jnp.exp(p2[0] * 10.0), 0.0))
    s1 = jnp.sum(jnp.where(sel, jnp.exp(p3[0] * 10.0), 0.0))

    @pl.when(m > bv_ref[0, 0])
    def _():
        bv_ref[0, 0] = m
        bi_ref[0, 0] = i * TILE_N + loc
        c0_ref[0, 0] = c0
        c1_ref[0, 0] = c1
        s0_ref[0, 0] = s0
        s1_ref[0, 0] = s1


def _b1(unc, pred):
    return pl.pallas_call(
        _b1_body,
        grid=(NT,),
        in_specs=[_img_spec()] + [_chan_spec(c) for c in (0, 1, 2, 3, 5, 6)],
        out_specs=[_scalar_out_spec()] * 6,
        out_shape=[_scalar_sd(jnp.float32), _scalar_sd(jnp.int32)]
        + [_scalar_sd(jnp.float32)] * 4,
    )(unc, *([pred] * 6))


def _b2_body(sv_ref, p0, p1, p2, p3, p4, p5, p6,
             prop_ref, n1_ref, bv_ref, bi_ref, c0_ref, c1_ref, s0_ref, s1_ref):
    i = pl.program_id(0)

    @pl.when(i == 0)
    def _():
        n1_ref[0, 0] = 0
        bv_ref[0, 0] = -1.0
        bi_ref[0, 0] = 0
        c0_ref[0, 0] = 0.0
        c1_ref[0, 0] = 0.0
        s0_ref[0, 0] = 0.0
        s1_ref[0, 0] = 0.0

    lidx, rows, cols = _lidx()
    e0, e1 = _emb(p0[0], p1[0], rows, cols, i)
    d0 = e0 - sv_ref[0]
    d1 = e1 - sv_ref[1]
    dist = jnp.exp(-(d0 * d0 * sv_ref[2] + d1 * d1 * sv_ref[3]))
    mask = p6[0] > p5[0]
    prop = (dist > DIST_THRESH) & mask
    prop_ref[...] = prop.astype(jnp.int32)
    n1_ref[0, 0] += jnp.sum(prop.astype(jnp.int32))

    sv = jnp.where(prop, jax.nn.sigmoid(p4[0]), 0.0)
    m = jnp.max(sv)
    loc = jnp.min(jnp.where(sv == m, lidx, BIG))
    sel = lidx == loc
    c0 = jnp.sum(jnp.where(sel, e0, 0.0))
    c1 = jnp.sum(jnp.where(sel, e1, 0.0))
    s0 = jnp.sum(jnp.where(sel, jnp.exp(p2[0] * 10.0), 0.0))
    s1 = jnp.sum(jnp.where(sel, jnp.exp(p3[0] * 10.0), 0.0))

    @pl.when(m > bv_ref[0, 0])
    def _():
        bv_ref[0, 0] = m
        bi_ref[0, 0] = i * TILE_N + loc
        c0_ref[0, 0] = c0
        c1_ref[0, 0] = c1
        s0_ref[0, 0] = s0
        s1_ref[0, 0] = s1


def _b2(center_sig, pred):
    return pl.pallas_call(
        _b2_body,
        grid=(NT,),
        in_specs=[_smem_spec()] + [_chan_spec(c) for c in range(7)],
        out_specs=[_img_spec()] + [_scalar_out_spec()] * 7,
        out_shape=[_img_sd(jnp.int32), _scalar_sd(jnp.int32),
                   _scalar_sd(jnp.float32), _scalar_sd(jnp.int32)]
        + [_scalar_sd(jnp.float32)] * 4,
    )(center_sig, *([pred] * 7))


def _b3_body(sv_ref, iv_ref, unc_ref, p0, p1, p5, p6,
             prop_ref, n2_ref, r_ref, cs1_ref, cs2_ref):
    i = pl.program_id(0)

    @pl.when(i == 0)
    def _():
        n2_ref[0, 0] = 0
        r_ref[0, 0] = 0
        cs1_ref[0, 0] = 0
        cs2_ref[0, 0] = 0

    lidx, rows, cols = _lidx()
    e0, e1 = _emb(p0[0], p1[0], rows, cols, i)
    d0 = e0 - sv_ref[0]
    d1 = e1 - sv_ref[1]
    dist = jnp.exp(-(d0 * d0 * sv_ref[2] + d1 * d1 * sv_ref[3]))
    mask = p6[0] > p5[0]
    prop = ((dist > DIST_THRESH) & mask).astype(jnp.int32)
    prop_ref[...] = prop
    u = unc_ref[...].astype(jnp.int32)
    pu = prop * u
    n2_ref[0, 0] += jnp.sum(prop)
    r_ref[0, 0] += jnp.sum(pu)
    gidx = lidx + i * TILE_N
    cs1_ref[0, 0] += jnp.sum(jnp.where(gidx == iv_ref[0], pu, 0))
    cs2_ref[0, 0] += jnp.sum(jnp.where(gidx == iv_ref[1], pu, 0))


def _b3(center_sig2, seeds, unc, pred):
    return pl.pallas_call(
        _b3_body,
        grid=(NT,),
        in_specs=[_smem_spec(), _smem_spec(), _img_spec()]
        + [_chan_spec(c) for c in (0, 1, 5, 6)],
        out_specs=[_img_spec()] + [_scalar_out_spec()] * 4,
        out_shape=[_img_sd(jnp.int32)] + [_scalar_sd(jnp.int32)] * 4,
    )(center_sig2, seeds, unc, *([pred] * 4))


def _b4_body(iv_ref, unc_ref, inst_ref, p1_ref, p2_ref,
             unc_out, inst_out, sum_ref):
    i = pl.program_id(0)

    @pl.when(i == 0)
    def _():
        sum_ref[0, 0] = 0

    seed = iv_ref[0]
    seed2 = iv_ref[1]
    count = iv_ref[2]
    broke = iv_ref[3] != 0
    big1 = iv_ref[4] != 0
    assign = iv_ref[5] != 0

    lidx, _, _ = _lidx()
    gidx = lidx + i * TILE_N
    u = unc_ref[...].astype(jnp.int32)
    u1 = jnp.where(gidx == seed, 0, u)
    u2 = jnp.where(gidx == seed2, 0, u1)
    prop1 = p1_ref[...]
    prop2 = p2_ref[...]
    fp = jnp.where(big1, prop2, prop1)
    umid = jnp.where(big1, u2, u1)
    unew = jnp.where(fp != 0, 0, umid)
    unew = jnp.where(broke, u, unew)
    inst = inst_ref[...].astype(jnp.int32)
    inew = jnp.where(assign & (prop2 != 0), count, inst)
    inew = jnp.where(broke, inst, inew)
    unc_out[...] = unew.astype(jnp.int8)
    inst_out[...] = inew.astype(jnp.uint8)
    sum_ref[0, 0] += jnp.sum(unew)


def _b4(ivec, unc, inst, prop1, prop2):
    return pl.pallas_call(
        _b4_body,
        grid=(NT,),
        in_specs=[_smem_spec()] + [_img_spec()] * 4,
        out_specs=[_img_spec(), _img_spec(), _scalar_out_spec()],
        out_shape=[_img_sd(jnp.int8), _img_sd(jnp.uint8),
                   _scalar_sd(jnp.int32)],
    )(ivec, unc, inst, prop1, prop2)


NW = 32
PER_W = (H * W) // NW
NB = 256


def _sc_hist_body(inst_hbm, out_hbm, buf_v, tab_v, res_v):
    c = jax.lax.axis_index("c")
    s = jax.lax.axis_index("s")
    wid = s * 2 + c
    z16 = jnp.zeros((16,), jnp.int32)
    for r in range(16):
        for g in range(NB // 16):
            tab_v[r, pl.ds(g * 16, 16)] = z16

    pltpu.sync_copy(inst_hbm.at[pl.ds(wid * PER_W, PER_W)], buf_v)
    lane = jax.lax.iota(jnp.int32, 16)
    ones = jnp.ones((16,), jnp.int32)

    def body(j, carry):
        v = plsc.load_gather(buf_v, [lane + j * 16])
        plsc.addupdate_scatter(tab_v, [lane, v], ones)
        return carry

    jax.lax.fori_loop(0, PER_W // 16, body, 0)

    for g in range(NB // 16):
        acc = z16
        for r in range(16):
            acc = acc + tab_v[r, pl.ds(g * 16, 16)]
        res_v[pl.ds(g * 16, 16)] = acc
    pltpu.sync_copy(res_v, out_hbm.at[wid])


def _tc_hist_body(inst_ref, now_ref):
    i = pl.program_id(0)

    @pl.when(i == 0)
    def _():
        now_ref[...] = jnp.zeros((1, NB), jnp.int32)

    t = inst_ref[...].astype(jnp.int32)
    lane = jax.lax.broadcasted_iota(jnp.int32, (1, NB), 1)

    def body(b, _):
        c = jnp.sum((t == b).astype(jnp.int32))
        now_ref[...] += jnp.where(lane == b, c, 0)
        return 0

    jax.lax.fori_loop(1, MAX_INST, body, 0)


def _tc_hist(inst):
    return pl.pallas_call(
        _tc_hist_body,
        grid=(NT,),
        in_specs=[_img_spec()],
        out_specs=pl.BlockSpec((1, NB), lambda i: (0, 0)),
        out_shape=jax.ShapeDtypeStruct((1, NB), jnp.int32),
    )(inst)


def _sc_hist(inst):
    mesh = plsc.VectorSubcoreMesh(core_axis_name="c", subcore_axis_name="s",
                                  num_cores=2, num_subcores=16)
    f = pl.kernel(
        _sc_hist_body,
        out_type=jax.ShapeDtypeStruct((NW, NB), jnp.int32),
        mesh=mesh,
        compiler_params=pltpu.CompilerParams(needs_layout_passes=False),
        scratch_types=[pltpu.VMEM((PER_W,), jnp.int32),
                       pltpu.VMEM((16, NB), jnp.int32),
                       pltpu.VMEM((NB,), jnp.int32)],
    )
    return f(inst.reshape(-1))


def _relabel_body(rm_ref, nrm_ref, inst_ref, out_ref):
    t = inst_ref[...].astype(jnp.int32)

    def body(j, acc):
        return jnp.where(t == rm_ref[j], 0, acc)

    res = jax.lax.fori_loop(0, nrm_ref[0], body, t)
    out_ref[...] = res.astype(jnp.uint8)


def _relabel(rm, nrm, inst):
    return pl.pallas_call(
        _relabel_body,
        grid=(NT,),
        in_specs=[_smem_spec(), _smem_spec(), _img_spec()],
        out_specs=_img_spec(),
        out_shape=_img_sd(jnp.uint8),
    )(rm, nrm, inst)


@functools.partial(jax.jit, static_argnames=())
def kernel(prediction):
    pred = prediction[0]

    unc0, inst0, cnt = _preamble(pred)
    unc_sum0 = jnp.sum(cnt)

    def cond_fn(state):
        unc, inst, sizes, count, done, unc_sum = state
        return (~done) & (unc_sum > MIN_PIXEL) & (count < MAX_INST)

    def body_fn(state):
        unc, inst, sizes, count, done, unc_sum = state
        bv, bi, c0, c1, s0, s1 = _b1(unc, pred)
        seed_score = bv[0, 0]
        seed = bi[0, 0]
        broke = seed_score < THRESH
        csig = jnp.stack([c0[0, 0], c1[0, 0], s0[0, 0], s1[0, 0]])
        prop1, n1r, bv2, bi2, c20, c21, s20, s21 = _b2(csig, pred)
        n1 = n1r[0, 0]
        big1 = n1 > MIN_INST_PIXEL
        seed2 = bi2[0, 0]
        csig2 = jnp.stack([c20[0, 0], c21[0, 0], s20[0, 0], s21[0, 0]])
        seeds = jnp.stack([seed, seed2])
        prop2, n2r, rr, cs1, cs2 = _b3(csig2, seeds, unc, pred)
        n2 = n2r[0, 0]
        big2 = n2 > MIN_INST_PIXEL
        inner = rr[0, 0] - cs1[0, 0] - jnp.where(seed2 != seed, cs2[0, 0], 0)
        ratio_ok = 2 * inner > n2
        assign = big1 & big2 & ratio_ok
        ivec = jnp.stack([seed, seed2, count,
                          broke.astype(jnp.int32), big1.astype(jnp.int32),
                          assign.astype(jnp.int32)])
        unc_new, inst_new, sum_new = _b4(ivec, unc, inst, prop1, prop2)
        keep = jnp.logical_and(assign, ~broke)
        sizes_new = jnp.where(keep, sizes.at[count].set(n2), sizes)
        count_new = count + jnp.where(keep, 1, 0)
        return (unc_new, inst_new, sizes_new, count_new, broke,
                sum_new[0, 0])

    state0 = (unc0, inst0, jnp.zeros((MAX_INST,), jnp.int32),
              jnp.int32(1), jnp.asarray(False), unc_sum0)
    unc, inst, sizes, count, done, unc_sum = jax.lax.while_loop(
        cond_fn, body_fn, state0)

    nowp = jax.lax.cond(count > 1, lambda: _tc_hist(inst),
                        lambda: jnp.zeros((1, NB), jnp.int32))
    now = nowp[0, :MAX_INST]
    prev = sizes
    remove = (now > 0) & (prev != now) & (
        (now < MIN_INST_PIXEL * 3) | (2 * now < prev))
    remove = remove.at[0].set(False)
    rm = jnp.sort(jnp.where(remove, jnp.arange(MAX_INST, dtype=jnp.int32),
                            MAX_INST))
    nrm = jnp.sum(remove.astype(jnp.int32)).reshape(1)
    out = _relabel(rm, nrm, inst)
    return out.reshape(1, H, W)


# speedup vs baseline: 2.1434x; 1.1410x over previous
import functools

import jax
import jax.numpy as jnp
from jax.experimental import pallas as pl
from jax.experimental.pallas import tpu as pltpu
from jax.experimental.pallas import tpu_sc as plsc

H, W = 1024, 2048
RT = 256
NT = H // RT
TILE_N = RT * W
XSTEP = 2.0 / (W - 1)
YSTEP = 1.0 / (H - 1)
BIG = 2 ** 30
THRESH = 0.5
DIST_THRESH = 0.5
MIN_PIXEL = 160
MIN_INST_PIXEL = 160
MAX_INST = 200


def _chan_spec(c):
    return pl.BlockSpec((1, RT, W), lambda i, c=c: (c, i, 0))


def _img_spec():
    return pl.BlockSpec((RT, W), lambda i: (i, 0))


def _scalar_out_spec():
    return pl.BlockSpec((1, 1), lambda i: (0, 0), memory_space=pltpu.SMEM)


def _smem_spec():
    return pl.BlockSpec(memory_space=pltpu.SMEM)


def _scalar_sd(dtype):
    return jax.ShapeDtypeStruct((1, 1), dtype)


def _img_sd(dtype):
    return jax.ShapeDtypeStruct((H, W), dtype)


def _lidx():
    rows = jax.lax.broadcasted_iota(jnp.int32, (RT, W), 0)
    cols = jax.lax.broadcasted_iota(jnp.int32, (RT, W), 1)
    return rows * W + cols, rows, cols


def _emb(p0, p1, rows, cols, tile):
    x = cols.astype(jnp.float32) * XSTEP
    y = (tile * RT + rows).astype(jnp.float32) * YSTEP
    return jnp.tanh(p0) + x, jnp.tanh(p1) + y


def _pre_body(p5_ref, p6_ref, unc_ref, inst_ref, cnt_ref):
    @pl.when(pl.program_id(0) == 0)
    def _():
        cnt_ref[0, 0] = 0

    m = p6_ref[0] > p5_ref[0]
    unc_ref[...] = m.astype(jnp.int8)
    inst_ref[...] = jnp.zeros((RT, W), jnp.uint8)
    cnt_ref[0, 0] += jnp.sum(m.astype(jnp.int32))


def _preamble(pred):
    return pl.pallas_call(
        _pre_body,
        grid=(NT,),
        in_specs=[_chan_spec(5), _chan_spec(6)],
        out_specs=[_img_spec(), _img_spec(), _scalar_out_spec()],
        out_shape=[_img_sd(jnp.int8), _img_sd(jnp.uint8),
                   _scalar_sd(jnp.int32)],
    )(pred, pred)


def _b1_body(unc_ref, p0, p1, p2, p3, p5, p6,
             bv_ref, bi_ref, c0_ref, c1_ref, s0_ref, s1_ref):
    i = pl.program_id(0)

    @pl.when(i == 0)
    def _():
        bv_ref[0, 0] = -1.0
        bi_ref[0, 0] = 0
        c0_ref[0, 0] = 0.0
        c1_ref[0, 0] = 0.0
        s0_ref[0, 0] = 0.0
        s1_ref[0, 0] = 0.0

    seedm = jax.nn.sigmoid(p6[0] - p5[0])
    scores = seedm * unc_ref[...].astype(jnp.float32)
    m = jnp.max(scores)
    lidx, rows, cols = _lidx()
    loc = jnp.min(jnp.where(scores == m, lidx, BIG))
    sel = lidx == loc
    e0, e1 = _emb(p0[0], p1[0], rows, cols, i)
    c0 = jnp.sum(jnp.where(sel, e0, 0.0))
    c1 = jnp.sum(jnp.where(sel, e1, 0.0))
    s0 = jnp.sum(jnp.where(sel, jnp.exp(p2[0] * 10.0), 0.0))
    s1 = jnp.sum(jnp.where(sel, jnp.exp(p3[0] * 10.0), 0.0))

    @pl.when(m > bv_ref[0, 0])
    def _():
        bv_ref[0, 0] = m
        bi_ref[0, 0] = i * TILE_N + loc
        c0_ref[0, 0] = c0
        c1_ref[0, 0] = c1
        s0_ref[0, 0] = s0
        s1_ref[0, 0] = s1


def _b1(unc, pred):
    return pl.pallas_call(
        _b1_body,
        grid=(NT,),
        in_specs=[_img_spec()] + [_chan_spec(c) for c in (0, 1, 2, 3, 5, 6)],
        out_specs=[_scalar_out_spec()] * 6,
        out_shape=[_scalar_sd(jnp.float32), _scalar_sd(jnp.int32)]
        + [_scalar_sd(jnp.float32)] * 4,
    )(unc, *([pred] * 6))


def _b2_body(sv_ref, p0, p1, p2, p3, p4, p5, p6,
             prop_ref, n1_ref, bv_ref, bi_ref, c0_ref, c1_ref, s0_ref, s1_ref):
    i = pl.program_id(0)

    @pl.when(i == 0)
    def _():
        n1_ref[0, 0] = 0
        bv_ref[0, 0] = -1.0
        bi_ref[0, 0] = 0
        c0_ref[0, 0] = 0.0
        c1_ref[0, 0] = 0.0
        s0_ref[0, 0] = 0.0
        s1_ref[0, 0] = 0.0

    lidx, rows, cols = _lidx()
    e0, e1 = _emb(p0[0], p1[0], rows, cols, i)
    d0 = e0 - sv_ref[0]
    d1 = e1 - sv_ref[1]
    dist = jnp.exp(-(d0 * d0 * sv_ref[2] + d1 * d1 * sv_ref[3]))
    mask = p6[0] > p5[0]
    prop = (dist > DIST_THRESH) & mask
    prop_ref[...] = prop.astype(jnp.int32)
    n1_ref[0, 0] += jnp.sum(prop.astype(jnp.int32))

    sv = jnp.where(prop, jax.nn.sigmoid(p4[0]), 0.0)
    m = jnp.max(sv)
    loc = jnp.min(jnp.where(sv == m, lidx, BIG))
    sel = lidx == loc
    c0 = jnp.sum(jnp.where(sel, e0, 0.0))
    c1 = jnp.sum(jnp.where(sel, e1, 0.0))
    s0 = jnp.sum(jnp.where(sel, jnp.exp(p2[0] * 10.0), 0.0))
    s1 = jnp.sum(jnp.where(sel, jnp.exp(p3[0] * 10.0), 0.0))

    @pl.when(m > bv_ref[0, 0])
    def _():
        bv_ref[0, 0] = m
        bi_ref[0, 0] = i * TILE_N + loc
        c0_ref[0, 0] = c0
        c1_ref[0, 0] = c1
        s0_ref[0, 0] = s0
        s1_ref[0, 0] = s1


def _b2(center_sig, pred):
    return pl.pallas_call(
        _b2_body,
        grid=(NT,),
        in_specs=[_smem_spec()] + [_chan_spec(c) for c in range(7)],
        out_specs=[_img_spec()] + [_scalar_out_spec()] * 7,
        out_shape=[_img_sd(jnp.int32), _scalar_sd(jnp.int32),
                   _scalar_sd(jnp.float32), _scalar_sd(jnp.int32)]
        + [_scalar_sd(jnp.float32)] * 4,
    )(center_sig, *([pred] * 7))


def _b3_body(sv_ref, iv_ref, unc_ref, p0, p1, p5, p6,
             prop_ref, n2_ref, r_ref, cs1_ref, cs2_ref):
    i = pl.program_id(0)

    @pl.when(i == 0)
    def _():
        n2_ref[0, 0] = 0
        r_ref[0, 0] = 0
        cs1_ref[0, 0] = 0
        cs2_ref[0, 0] = 0

    lidx, rows, cols = _lidx()
    e0, e1 = _emb(p0[0], p1[0], rows, cols, i)
    d0 = e0 - sv_ref[0]
    d1 = e1 - sv_ref[1]
    dist = jnp.exp(-(d0 * d0 * sv_ref[2] + d1 * d1 * sv_ref[3]))
    mask = p6[0] > p5[0]
    prop = ((dist > DIST_THRESH) & mask).astype(jnp.int32)
    prop_ref[...] = prop
    u = unc_ref[...].astype(jnp.int32)
    pu = prop * u
    n2_ref[0, 0] += jnp.sum(prop)
    r_ref[0, 0] += jnp.sum(pu)
    gidx = lidx + i * TILE_N
    cs1_ref[0, 0] += jnp.sum(jnp.where(gidx == iv_ref[0], pu, 0))
    cs2_ref[0, 0] += jnp.sum(jnp.where(gidx == iv_ref[1], pu, 0))


def _b3(center_sig2, seeds, unc, pred):
    return pl.pallas_call(
        _b3_body,
        grid=(NT,),
        in_specs=[_smem_spec(), _smem_spec(), _img_spec()]
        + [_chan_spec(c) for c in (0, 1, 5, 6)],
        out_specs=[_img_spec()] + [_scalar_out_spec()] * 4,
        out_shape=[_img_sd(jnp.int32)] + [_scalar_sd(jnp.int32)] * 4,
    )(center_sig2, seeds, unc, *([pred] * 4))


def _b4_body(iv_ref, unc_ref, inst_ref, p1_ref, p2_ref,
             unc_out, inst_out, sum_ref):
    i = pl.program_id(0)

    @pl.when(i == 0)
    def _():
        sum_ref[0, 0] = 0

    seed = iv_ref[0]
    seed2 = iv_ref[1]
    count = iv_ref[2]
    broke = iv_ref[3] != 0
    big1 = iv_ref[4] != 0
    assign = iv_ref[5] != 0

    lidx, _, _ = _lidx()
    gidx = lidx + i * TILE_N
    u = unc_ref[...].astype(jnp.int32)
    u1 = jnp.where(gidx == seed, 0, u)
    u2 = jnp.where(gidx == seed2, 0, u1)
    prop1 = p1_ref[...]
    prop2 = p2_ref[...]
    fp = jnp.where(big1, prop2, prop1)
    umid = jnp.where(big1, u2, u1)
    unew = jnp.where(fp != 0, 0, umid)
    unew = jnp.where(broke, u, unew)
    inst = inst_ref[...].astype(jnp.int32)
    inew = jnp.where(assign & (prop2 != 0), count, inst)
    inew = jnp.where(broke, inst, inew)
    unc_out[...] = unew.astype(jnp.int8)
    inst_out[...] = inew.astype(jnp.uint8)
    sum_ref[0, 0] += jnp.sum(unew)


def _b4(ivec, unc, inst, prop1, prop2):
    return pl.pallas_call(
        _b4_body,
        grid=(NT,),
        in_specs=[_smem_spec()] + [_img_spec()] * 4,
        out_specs=[_img_spec(), _img_spec(), _scalar_out_spec()],
        out_shape=[_img_sd(jnp.int8), _img_sd(jnp.uint8),
                   _scalar_sd(jnp.int32)],
    )(ivec, unc, inst, prop1, prop2)


NW = 32
PER_W = (H * W) // NW
NB = 256


def _sc_hist_body(inst_hbm, out_hbm, buf_v, tab_v, res_v):
    c = jax.lax.axis_index("c")
    s = jax.lax.axis_index("s")
    wid = s * 2 + c
    z16 = jnp.zeros((16,), jnp.int32)
    for r in range(16):
        for g in range(NB // 16):
            tab_v[r, pl.ds(g * 16, 16)] = z16

    pltpu.sync_copy(inst_hbm.at[pl.ds(wid * PER_W, PER_W)], buf_v)
    lane = jax.lax.iota(jnp.int32, 16)
    ones = jnp.ones((16,), jnp.int32)

    def body(j, carry):
        v = plsc.load_gather(buf_v, [lane + j * 16])
        plsc.addupdate_scatter(tab_v, [lane, v], ones)
        return carry

    jax.lax.fori_loop(0, PER_W // 16, body, 0)

    for g in range(NB // 16):
        acc = z16
        for r in range(16):
            acc = acc + tab_v[r, pl.ds(g * 16, 16)]
        res_v[pl.ds(g * 16, 16)] = acc
    pltpu.sync_copy(res_v, out_hbm.at[wid])


def _tc_hist_body(inst_ref, now_ref):
    i = pl.program_id(0)

    @pl.when(i == 0)
    def _():
        now_ref[...] = jnp.zeros((1, NB), jnp.int32)

    t = inst_ref[...].astype(jnp.int32)
    lane = jax.lax.broadcasted_iota(jnp.int32, (1, NB), 1)

    def body(b, _):
        c = jnp.sum((t == b).astype(jnp.int32))
        now_ref[...] += jnp.where(lane == b, c, 0)
        return 0

    jax.lax.fori_loop(1, MAX_INST, body, 0)


def _tc_hist(inst):
    return pl.pallas_call(
        _tc_hist_body,
        grid=(NT,),
        in_specs=[_img_spec()],
        out_specs=pl.BlockSpec((1, NB), lambda i: (0, 0)),
        out_shape=jax.ShapeDtypeStruct((1, NB), jnp.int32),
    )(inst)


def _sc_hist(inst):
    mesh = plsc.VectorSubcoreMesh(core_axis_name="c", subcore_axis_name="s",
                                  num_cores=2, num_subcores=16)
    f = pl.kernel(
        _sc_hist_body,
        out_type=jax.ShapeDtypeStruct((NW, NB), jnp.int32),
        mesh=mesh,
        compiler_params=pltpu.CompilerParams(needs_layout_passes=False),
        scratch_types=[pltpu.VMEM((PER_W,), jnp.int32),
                       pltpu.VMEM((16, NB), jnp.int32),
                       pltpu.VMEM((NB,), jnp.int32)],
    )
    return f(inst.reshape(-1))


def _relabel_body(rm_ref, nrm_ref, inst_ref, out_ref):
    t = inst_ref[...].astype(jnp.int32)

    def body(j, acc):
        return jnp.where(t == rm_ref[j], 0, acc)

    res = jax.lax.fori_loop(0, nrm_ref[0], body, t)
    out_ref[...] = res.astype(jnp.uint8)


def _relabel(rm, nrm, inst):
    return pl.pallas_call(
        _relabel_body,
        grid=(NT,),
        in_specs=[_smem_spec(), _smem_spec(), _img_spec()],
        out_specs=_img_spec(),
        out_shape=_img_sd(jnp.uint8),
    )(rm, nrm, inst)


@functools.partial(jax.jit, static_argnames=())
def kernel(prediction):
    pred = prediction[0]

    unc0, inst0, cnt = _preamble(pred)
    unc_sum0 = jnp.sum(cnt)

    def cond_fn(state):
        unc, inst, sizes, count, done, unc_sum = state
        return (~done) & (unc_sum > MIN_PIXEL) & (count < MAX_INST)

    def body_fn(state):
        unc, inst, sizes, count, done, unc_sum = state
        bv, bi, c0, c1, s0, s1 = _b1(unc, pred)
        seed_score = bv[0, 0]
        seed = bi[0, 0]
        broke = seed_score < THRESH
        csig = jnp.stack([c0[0, 0], c1[0, 0], s0[0, 0], s1[0, 0]])
        prop1, n1r, bv2, bi2, c20, c21, s20, s21 = _b2(csig, pred)
        n1 = n1r[0, 0]
        big1 = n1 > MIN_INST_PIXEL
        seed2 = bi2[0, 0]
        csig2 = jnp.stack([c20[0, 0], c21[0, 0], s20[0, 0], s21[0, 0]])
        seeds = jnp.stack([seed, seed2])
        prop2, n2r, rr, cs1, cs2 = _b3(csig2, seeds, unc, pred)
        n2 = n2r[0, 0]
        big2 = n2 > MIN_INST_PIXEL
        inner = rr[0, 0] - cs1[0, 0] - jnp.where(seed2 != seed, cs2[0, 0], 0)
        ratio_ok = 2 * inner > n2
        assign = big1 & big2 & ratio_ok
        ivec = jnp.stack([seed, seed2, count,
                          broke.astype(jnp.int32), big1.astype(jnp.int32),
                          assign.astype(jnp.int32)])
        unc_new, inst_new, sum_new = _b4(ivec, unc, inst, prop1, prop2)
        keep = jnp.logical_and(assign, ~broke)
        sizes_new = jnp.where(keep, sizes.at[count].set(n2), sizes)
        count_new = count + jnp.where(keep, 1, 0)
        return (unc_new, inst_new, sizes_new, count_new, broke,
                sum_new[0, 0])

    state0 = (unc0, inst0, jnp.zeros((MAX_INST,), jnp.int32),
              jnp.int32(1), jnp.asarray(False), unc_sum0)
    unc, inst, sizes, count, done, unc_sum = jax.lax.while_loop(
        cond_fn, body_fn, state0)

    nowp = jax.lax.cond(count > 1, lambda: _tc_hist(inst),
                        lambda: jnp.zeros((1, NB), jnp.int32))
    now = nowp[0, :MAX_INST]
    prev = sizes
    remove = (now > 0) & (prev != now) & (
        (now < MIN_INST_PIXEL * 3) | (2 * now < prev))
    remove = remove.at[0].set(False)
    rm = jnp.sort(jnp.where(remove, jnp.arange(MAX_INST, dtype=jnp.int32),
                            MAX_INST))
    nrm = jnp.sum(remove.astype(jnp.int32)).reshape(1)
    out = _relabel(rm, nrm, inst)
    return out.reshape(1, H, W)


# speedup vs baseline: 2.2400x; 1.0451x over previous
import functools

import jax
import jax.numpy as jnp
from jax.experimental import pallas as pl
from jax.experimental.pallas import tpu as pltpu
from jax.experimental.pallas import tpu_sc as plsc

H, W = 1024, 2048
RT = 256
NT = H // RT
RT_L = 512
NT_L = H // RT_L
TILE_N = RT * W
XSTEP = 2.0 / (W - 1)
YSTEP = 1.0 / (H - 1)
BIG = 2 ** 30
THRESH = 0.5
DIST_THRESH = 0.5
MIN_PIXEL = 160
MIN_INST_PIXEL = 160
MAX_INST = 200


def _chan_spec(c):
    return pl.BlockSpec((1, RT, W), lambda i, c=c: (c, i, 0))


def _img_spec():
    return pl.BlockSpec((RT, W), lambda i: (i, 0))


def _scalar_out_spec():
    return pl.BlockSpec((1, 1), lambda i: (0, 0), memory_space=pltpu.SMEM)


def _smem_spec():
    return pl.BlockSpec(memory_space=pltpu.SMEM)


def _chan_spec_l(c):
    return pl.BlockSpec((1, RT_L, W), lambda i, c=c: (c, i, 0))


def _img_spec_l():
    return pl.BlockSpec((RT_L, W), lambda i: (i, 0))


def _scalar_sd(dtype):
    return jax.ShapeDtypeStruct((1, 1), dtype)


def _img_sd(dtype):
    return jax.ShapeDtypeStruct((H, W), dtype)


def _lidx():
    rows = jax.lax.broadcasted_iota(jnp.int32, (RT, W), 0)
    cols = jax.lax.broadcasted_iota(jnp.int32, (RT, W), 1)
    return rows * W + cols, rows, cols


def _emb(p0, p1, rows, cols, tile):
    x = cols.astype(jnp.float32) * XSTEP
    y = (tile * RT + rows).astype(jnp.float32) * YSTEP
    return jnp.tanh(p0) + x, jnp.tanh(p1) + y


def _pre_body(p5_ref, p6_ref, unc_ref, inst_ref, cnt_ref):
    @pl.when(pl.program_id(0) == 0)
    def _():
        cnt_ref[0, 0] = 0

    m = p6_ref[0] > p5_ref[0]
    unc_ref[...] = m.astype(jnp.int8)
    inst_ref[...] = jnp.zeros((RT_L, W), jnp.uint8)
    cnt_ref[0, 0] += jnp.sum(m.astype(jnp.int32))


def _preamble(pred):
    return pl.pallas_call(
        _pre_body,
        grid=(NT_L,),
        in_specs=[_chan_spec_l(5), _chan_spec_l(6)],
        out_specs=[_img_spec_l(), _img_spec_l(), _scalar_out_spec()],
        out_shape=[_img_sd(jnp.int8), _img_sd(jnp.uint8),
                   _scalar_sd(jnp.int32)],
    )(pred, pred)


def _b1_body(unc_ref, p0, p1, p2, p3, p5, p6,
             bv_ref, bi_ref, c0_ref, c1_ref, s0_ref, s1_ref):
    i = pl.program_id(0)

    @pl.when(i == 0)
    def _():
        bv_ref[0, 0] = -1.0
        bi_ref[0, 0] = 0
        c0_ref[0, 0] = 0.0
        c1_ref[0, 0] = 0.0
        s0_ref[0, 0] = 0.0
        s1_ref[0, 0] = 0.0

    seedm = jax.nn.sigmoid(p6[0] - p5[0])
    scores = seedm * unc_ref[...].astype(jnp.float32)
    m = jnp.max(scores)
    lidx, rows, cols = _lidx()
    loc = jnp.min(jnp.where(scores == m, lidx, BIG))
    sel = lidx == loc
    e0, e1 = _emb(p0[0], p1[0], rows, cols, i)
    c0 = jnp.sum(jnp.where(sel, e0, 0.0))
    c1 = jnp.sum(jnp.where(sel, e1, 0.0))
    s0 = jnp.sum(jnp.where(sel, jnp.exp(p2[0] * 10.0), 0.0))
    s1 = jnp.sum(jnp.where(sel, jnp.exp(p3[0] * 10.0), 0.0))

    @pl.when(m > bv_ref[0, 0])
    def _():
        bv_ref[0, 0] = m
        bi_ref[0, 0] = i * TILE_N + loc
        c0_ref[0, 0] = c0
        c1_ref[0, 0] = c1
        s0_ref[0, 0] = s0
        s1_ref[0, 0] = s1


def _b1(unc, pred):
    return pl.pallas_call(
        _b1_body,
        grid=(NT,),
        in_specs=[_img_spec()] + [_chan_spec(c) for c in (0, 1, 2, 3, 5, 6)],
        out_specs=[_scalar_out_spec()] * 6,
        out_shape=[_scalar_sd(jnp.float32), _scalar_sd(jnp.int32)]
        + [_scalar_sd(jnp.float32)] * 4,
    )(unc, *([pred] * 6))


def _b2_body(sv_ref, p0, p1, p2, p3, p4, p5, p6,
             prop_ref, n1_ref, bv_ref, bi_ref, c0_ref, c1_ref, s0_ref, s1_ref):
    i = pl.program_id(0)

    @pl.when(i == 0)
    def _():
        n1_ref[0, 0] = 0
        bv_ref[0, 0] = -1.0
        bi_ref[0, 0] = 0
        c0_ref[0, 0] = 0.0
        c1_ref[0, 0] = 0.0
        s0_ref[0, 0] = 0.0
        s1_ref[0, 0] = 0.0

    lidx, rows, cols = _lidx()
    e0, e1 = _emb(p0[0], p1[0], rows, cols, i)
    d0 = e0 - sv_ref[0]
    d1 = e1 - sv_ref[1]
    dist = jnp.exp(-(d0 * d0 * sv_ref[2] + d1 * d1 * sv_ref[3]))
    mask = p6[0] > p5[0]
    prop = (dist > DIST_THRESH) & mask
    prop_ref[...] = prop.astype(jnp.int32)
    n1_ref[0, 0] += jnp.sum(prop.astype(jnp.int32))

    sv = jnp.where(prop, jax.nn.sigmoid(p4[0]), 0.0)
    m = jnp.max(sv)
    loc = jnp.min(jnp.where(sv == m, lidx, BIG))
    sel = lidx == loc
    c0 = jnp.sum(jnp.where(sel, e0, 0.0))
    c1 = jnp.sum(jnp.where(sel, e1, 0.0))
    s0 = jnp.sum(jnp.where(sel, jnp.exp(p2[0] * 10.0), 0.0))
    s1 = jnp.sum(jnp.where(sel, jnp.exp(p3[0] * 10.0), 0.0))

    @pl.when(m > bv_ref[0, 0])
    def _():
        bv_ref[0, 0] = m
        bi_ref[0, 0] = i * TILE_N + loc
        c0_ref[0, 0] = c0
        c1_ref[0, 0] = c1
        s0_ref[0, 0] = s0
        s1_ref[0, 0] = s1


def _b2(center_sig, pred):
    return pl.pallas_call(
        _b2_body,
        grid=(NT,),
        in_specs=[_smem_spec()] + [_chan_spec(c) for c in range(7)],
        out_specs=[_img_spec()] + [_scalar_out_spec()] * 7,
        out_shape=[_img_sd(jnp.int32), _scalar_sd(jnp.int32),
                   _scalar_sd(jnp.float32), _scalar_sd(jnp.int32)]
        + [_scalar_sd(jnp.float32)] * 4,
    )(center_sig, *([pred] * 7))


def _b3_body(sv_ref, iv_ref, unc_ref, p0, p1, p5, p6,
             prop_ref, n2_ref, r_ref, cs1_ref, cs2_ref):
    i = pl.program_id(0)

    @pl.when(i == 0)
    def _():
        n2_ref[0, 0] = 0
        r_ref[0, 0] = 0
        cs1_ref[0, 0] = 0
        cs2_ref[0, 0] = 0

    lidx, rows, cols = _lidx()
    e0, e1 = _emb(p0[0], p1[0], rows, cols, i)
    d0 = e0 - sv_ref[0]
    d1 = e1 - sv_ref[1]
    dist = jnp.exp(-(d0 * d0 * sv_ref[2] + d1 * d1 * sv_ref[3]))
    mask = p6[0] > p5[0]
    prop = ((dist > DIST_THRESH) & mask).astype(jnp.int32)
    prop_ref[...] = prop
    u = unc_ref[...].astype(jnp.int32)
    pu = prop * u
    n2_ref[0, 0] += jnp.sum(prop)
    r_ref[0, 0] += jnp.sum(pu)
    gidx = lidx + i * TILE_N
    cs1_ref[0, 0] += jnp.sum(jnp.where(gidx == iv_ref[0], pu, 0))
    cs2_ref[0, 0] += jnp.sum(jnp.where(gidx == iv_ref[1], pu, 0))


def _b3(center_sig2, seeds, unc, pred):
    return pl.pallas_call(
        _b3_body,
        grid=(NT,),
        in_specs=[_smem_spec(), _smem_spec(), _img_spec()]
        + [_chan_spec(c) for c in (0, 1, 5, 6)],
        out_specs=[_img_spec()] + [_scalar_out_spec()] * 4,
        out_shape=[_img_sd(jnp.int32)] + [_scalar_sd(jnp.int32)] * 4,
    )(center_sig2, seeds, unc, *([pred] * 4))


def _b4_body(iv_ref, unc_ref, inst_ref, p1_ref, p2_ref,
             unc_out, inst_out, sum_ref):
    i = pl.program_id(0)

    @pl.when(i == 0)
    def _():
        sum_ref[0, 0] = 0

    seed = iv_ref[0]
    seed2 = iv_ref[1]
    count = iv_ref[2]
    broke = iv_ref[3] != 0
    big1 = iv_ref[4] != 0
    assign = iv_ref[5] != 0

    lidx, _, _ = _lidx()
    gidx = lidx + i * TILE_N
    u = unc_ref[...].astype(jnp.int32)
    u1 = jnp.where(gidx == seed, 0, u)
    u2 = jnp.where(gidx == seed2, 0, u1)
    prop1 = p1_ref[...]
    prop2 = p2_ref[...]
    fp = jnp.where(big1, prop2, prop1)
    umid = jnp.where(big1, u2, u1)
    unew = jnp.where(fp != 0, 0, umid)
    unew = jnp.where(broke, u, unew)
    inst = inst_ref[...].astype(jnp.int32)
    inew = jnp.where(assign & (prop2 != 0), count, inst)
    inew = jnp.where(broke, inst, inew)
    unc_out[...] = unew.astype(jnp.int8)
    inst_out[...] = inew.astype(jnp.uint8)
    sum_ref[0, 0] += jnp.sum(unew)


def _b4(ivec, unc, inst, prop1, prop2):
    return pl.pallas_call(
        _b4_body,
        grid=(NT,),
        in_specs=[_smem_spec()] + [_img_spec()] * 4,
        out_specs=[_img_spec(), _img_spec(), _scalar_out_spec()],
        out_shape=[_img_sd(jnp.int8), _img_sd(jnp.uint8),
                   _scalar_sd(jnp.int32)],
    )(ivec, unc, inst, prop1, prop2)


NW = 32
PER_W = (H * W) // NW
NB = 256


def _sc_hist_body(inst_hbm, out_hbm, buf_v, tab_v, res_v):
    c = jax.lax.axis_index("c")
    s = jax.lax.axis_index("s")
    wid = s * 2 + c
    z16 = jnp.zeros((16,), jnp.int32)
    for r in range(16):
        for g in range(NB // 16):
            tab_v[r, pl.ds(g * 16, 16)] = z16

    pltpu.sync_copy(inst_hbm.at[pl.ds(wid * PER_W, PER_W)], buf_v)
    lane = jax.lax.iota(jnp.int32, 16)
    ones = jnp.ones((16,), jnp.int32)

    def body(j, carry):
        v = plsc.load_gather(buf_v, [lane + j * 16])
        plsc.addupdate_scatter(tab_v, [lane, v], ones)
        return carry

    jax.lax.fori_loop(0, PER_W // 16, body, 0)

    for g in range(NB // 16):
        acc = z16
        for r in range(16):
            acc = acc + tab_v[r, pl.ds(g * 16, 16)]
        res_v[pl.ds(g * 16, 16)] = acc
    pltpu.sync_copy(res_v, out_hbm.at[wid])


def _tc_hist_body(inst_ref, now_ref):
    i = pl.program_id(0)

    @pl.when(i == 0)
    def _():
        now_ref[...] = jnp.zeros((1, NB), jnp.int32)

    t = inst_ref[...].astype(jnp.int32)
    lane = jax.lax.broadcasted_iota(jnp.int32, (1, NB), 1)

    def body(b, _):
        c = jnp.sum((t == b).astype(jnp.int32))
        now_ref[...] += jnp.where(lane == b, c, 0)
        return 0

    jax.lax.fori_loop(1, MAX_INST, body, 0)


def _tc_hist(inst):
    return pl.pallas_call(
        _tc_hist_body,
        grid=(NT_L,),
        in_specs=[_img_spec_l()],
        out_specs=pl.BlockSpec((1, NB), lambda i: (0, 0)),
        out_shape=jax.ShapeDtypeStruct((1, NB), jnp.int32),
    )(inst)


def _sc_hist(inst):
    mesh = plsc.VectorSubcoreMesh(core_axis_name="c", subcore_axis_name="s",
                                  num_cores=2, num_subcores=16)
    f = pl.kernel(
        _sc_hist_body,
        out_type=jax.ShapeDtypeStruct((NW, NB), jnp.int32),
        mesh=mesh,
        compiler_params=pltpu.CompilerParams(needs_layout_passes=False),
        scratch_types=[pltpu.VMEM((PER_W,), jnp.int32),
                       pltpu.VMEM((16, NB), jnp.int32),
                       pltpu.VMEM((NB,), jnp.int32)],
    )
    return f(inst.reshape(-1))


def _relabel_body(rm_ref, nrm_ref, inst_ref, out_ref):
    t = inst_ref[...].astype(jnp.int32)

    def body(j, acc):
        return jnp.where(t == rm_ref[j], 0, acc)

    res = jax.lax.fori_loop(0, nrm_ref[0], body, t)
    out_ref[...] = res.astype(jnp.uint8)


def _relabel(rm, nrm, inst):
    return pl.pallas_call(
        _relabel_body,
        grid=(NT_L,),
        in_specs=[_smem_spec(), _smem_spec(), _img_spec_l()],
        out_specs=_img_spec_l(),
        out_shape=_img_sd(jnp.uint8),
    )(rm, nrm, inst)


@functools.partial(jax.jit, static_argnames=())
def kernel(prediction):
    pred = prediction[0]

    unc0, inst0, cnt = _preamble(pred)
    unc_sum0 = jnp.sum(cnt)

    def cond_fn(state):
        unc, inst, sizes, count, done, unc_sum = state
        return (~done) & (unc_sum > MIN_PIXEL) & (count < MAX_INST)

    def body_fn(state):
        unc, inst, sizes, count, done, unc_sum = state
        bv, bi, c0, c1, s0, s1 = _b1(unc, pred)
        seed_score = bv[0, 0]
        seed = bi[0, 0]
        broke = seed_score < THRESH
        csig = jnp.stack([c0[0, 0], c1[0, 0], s0[0, 0], s1[0, 0]])
        prop1, n1r, bv2, bi2, c20, c21, s20, s21 = _b2(csig, pred)
        n1 = n1r[0, 0]
        big1 = n1 > MIN_INST_PIXEL
        seed2 = bi2[0, 0]
        csig2 = jnp.stack([c20[0, 0], c21[0, 0], s20[0, 0], s21[0, 0]])
        seeds = jnp.stack([seed, seed2])
        prop2, n2r, rr, cs1, cs2 = _b3(csig2, seeds, unc, pred)
        n2 = n2r[0, 0]
        big2 = n2 > MIN_INST_PIXEL
        inner = rr[0, 0] - cs1[0, 0] - jnp.where(seed2 != seed, cs2[0, 0], 0)
        ratio_ok = 2 * inner > n2
        assign = big1 & big2 & ratio_ok
        ivec = jnp.stack([seed, seed2, count,
                          broke.astype(jnp.int32), big1.astype(jnp.int32),
                          assign.astype(jnp.int32)])
        unc_new, inst_new, sum_new = _b4(ivec, unc, inst, prop1, prop2)
        keep = jnp.logical_and(assign, ~broke)
        sizes_new = jnp.where(keep, sizes.at[count].set(n2), sizes)
        count_new = count + jnp.where(keep, 1, 0)
        return (unc_new, inst_new, sizes_new, count_new, broke,
                sum_new[0, 0])

    state0 = (unc0, inst0, jnp.zeros((MAX_INST,), jnp.int32),
              jnp.int32(1), jnp.asarray(False), unc_sum0)
    unc, inst, sizes, count, done, unc_sum = jax.lax.while_loop(
        cond_fn, body_fn, state0)

    nowp = jax.lax.cond(count > 1, lambda: _tc_hist(inst),
                        lambda: jnp.zeros((1, NB), jnp.int32))
    now = nowp[0, :MAX_INST]
    prev = sizes
    remove = (now > 0) & (prev != now) & (
        (now < MIN_INST_PIXEL * 3) | (2 * now < prev))
    remove = remove.at[0].set(False)
    rm = jnp.sort(jnp.where(remove, jnp.arange(MAX_INST, dtype=jnp.int32),
                            MAX_INST))
    nrm = jnp.sum(remove.astype(jnp.int32)).reshape(1)
    out = _relabel(rm, nrm, inst)
    return out.reshape(1, H, W)
